# baseline (speedup 1.0000x reference)
import jax
import jax.numpy as jnp
from jax import lax
from jax.experimental import pallas as pl
from jax.experimental.pallas import tpu as pltpu

BATCH = 1024
IN_FEATURES = 128
OUT_FEATURES = 128
GRID_SIZE = 8
MIN_VALUE = -2.0
MAX_VALUE = 2.0

BLOCK_B = 256
K = IN_FEATURES * GRID_SIZE


def _kan_kernel(x_ref, basis_ref, bias_ref, out_ref):
    x = x_ref[:]
    scaled = (jnp.clip(x, MIN_VALUE, MAX_VALUE) - MIN_VALUE) * (
        (GRID_SIZE - 1) / (MAX_VALUE - MIN_VALUE)
    )
    rows = lax.broadcasted_iota(jnp.int32, (IN_FEATURES, K), 0)
    cols = lax.broadcasted_iota(jnp.int32, (IN_FEATURES, K), 1) >> 3
    rep_mat = (rows == cols).astype(jnp.float32)
    w_pre = jnp.dot(scaled, rep_mat, preferred_element_type=jnp.float32)
    gi = lax.broadcasted_iota(jnp.int32, (BLOCK_B, K), 1) & (GRID_SIZE - 1)
    g = gi.astype(jnp.float32)
    w = jnp.maximum(1.0 - jnp.abs(w_pre - g), 0.0)
    out = lax.dot_general(
        w, basis_ref[:],
        (((1,), (1,)), ((), ())),
        preferred_element_type=jnp.float32,
    )
    out_ref[:] = out + bias_ref[:]


def kernel(inputs, basis, bias):
    basis_flat = basis.reshape(OUT_FEATURES, K)
    bias2d = bias.reshape(1, OUT_FEATURES)
    grid = (BATCH // BLOCK_B,)
    return pl.pallas_call(
        _kan_kernel,
        grid=grid,
        in_specs=[
            pl.BlockSpec((BLOCK_B, IN_FEATURES), lambda i: (i, 0)),
            pl.BlockSpec((OUT_FEATURES, K), lambda i: (0, 0)),
            pl.BlockSpec((1, OUT_FEATURES), lambda i: (0, 0)),
        ],
        out_specs=pl.BlockSpec((BLOCK_B, OUT_FEATURES), lambda i: (i, 0)),
        out_shape=jax.ShapeDtypeStruct((BATCH, OUT_FEATURES), jnp.float32),
    )(inputs, basis_flat, bias2d)
